# baseline (speedup 1.0000x reference)
import jax
import jax.numpy as jnp
from jax.experimental import pallas as pl
from jax.experimental.pallas import tpu as pltpu

K = 8
B = 1024
SUBD = 64
AUG = SUBD + 8
BV = 4096


def _prep_kernel(cb_ref, cbs_ref):
    cb = cb_ref[...]
    cbt = jnp.swapaxes(cb, 1, 2)
    c2 = jnp.sum(cbt * cbt, axis=1)
    hi = c2.astype(jnp.bfloat16).astype(jnp.float32)
    r1 = c2 - hi
    mid = r1.astype(jnp.bfloat16).astype(jnp.float32)
    lo = (r1 - mid).astype(jnp.bfloat16).astype(jnp.float32)
    cbs_ref[:, :SUBD, :] = cbt * -2.0
    cbs_ref[:, SUBD:SUBD + 3, :] = jnp.stack([hi, mid, lo], axis=1)
    cbs_ref[:, SUBD + 3:, :] = jnp.zeros((K, 5, B), jnp.float32)


def _pq_kernel(x_ref, cbs_ref, out_ref):
    one = jnp.ones((BV, 1), jnp.float32)
    zero = jnp.zeros((BV, 1), jnp.float32)
    aug = jnp.concatenate([one, one, one, zero, zero, zero, zero, zero],
                          axis=1)
    cols = []
    for k in range(K):
        xa = jnp.concatenate(
            [x_ref[:, k * SUBD:(k + 1) * SUBD], aug], axis=1)
        d2 = jax.lax.dot(
            xa, cbs_ref[k],
            precision=jax.lax.Precision.DEFAULT,
            preferred_element_type=jnp.float32,
        )
        idx = jnp.argmin(d2, axis=1, keepdims=True).astype(jnp.int32)
        cols.append(idx)
    out_ref[:, :] = jnp.concatenate(cols, axis=1).astype(jnp.uint8)


@jax.jit
def kernel(vectors, codebook):
    v, _ = vectors.shape
    cbs = pl.pallas_call(
        _prep_kernel,
        out_shape=jax.ShapeDtypeStruct((K, AUG, B), jnp.float32),
    )(codebook)
    out = pl.pallas_call(
        _pq_kernel,
        grid=(v // BV,),
        in_specs=[
            pl.BlockSpec((BV, K * SUBD), lambda i: (i, 0)),
            pl.BlockSpec((K, AUG, B), lambda i: (0, 0, 0)),
        ],
        out_specs=pl.BlockSpec((BV, K), lambda i: (i, 0)),
        out_shape=jax.ShapeDtypeStruct((v, K), jnp.uint8),
        compiler_params=pltpu.CompilerParams(
            dimension_semantics=("parallel",),
        ),
    )(vectors, cbs)
    return out


# speedup vs baseline: 1.0117x; 1.0117x over previous
import jax
import jax.numpy as jnp
from jax.experimental import pallas as pl
from jax.experimental.pallas import tpu as pltpu

K = 8
B = 1024
SUBD = 64
AUG = SUBD + 4
BV = 4096


def _prep_kernel(cbt_ref, cbs_ref):
    cbt = cbt_ref[...]
    c2 = jnp.sum(cbt * cbt, axis=1)
    hi = c2.astype(jnp.bfloat16).astype(jnp.float32)
    r1 = c2 - hi
    mid = r1.astype(jnp.bfloat16).astype(jnp.float32)
    lo = (r1 - mid).astype(jnp.bfloat16).astype(jnp.float32)
    cbs_ref[:, :SUBD, :] = cbt * -2.0
    cbs_ref[:, SUBD:SUBD + 3, :] = jnp.stack([hi, mid, lo], axis=1)
    cbs_ref[:, SUBD + 3:, :] = jnp.zeros((K, AUG - SUBD - 3, B), jnp.float32)


def _pq_kernel(x_ref, cbs_ref, out_ref):
    one = jnp.ones((BV, 1), jnp.float32)
    zero = jnp.zeros((BV, 1), jnp.float32)
    aug = jnp.concatenate([one, one, one, zero], axis=1)
    cols = []
    for k in range(K):
        xa = jnp.concatenate(
            [x_ref[:, k * SUBD:(k + 1) * SUBD], aug], axis=1)
        d2 = jax.lax.dot(
            xa, cbs_ref[k],
            precision=jax.lax.Precision.DEFAULT,
            preferred_element_type=jnp.float32,
        )
        idx = jnp.argmin(d2, axis=1, keepdims=True).astype(jnp.int32)
        cols.append(idx)
    out_ref[:, :] = jnp.concatenate(cols, axis=1).astype(jnp.uint8)


@jax.jit
def kernel(vectors, codebook):
    v, _ = vectors.shape
    cbt = jnp.swapaxes(codebook, 1, 2)
    cbs = pl.pallas_call(
        _prep_kernel,
        out_shape=jax.ShapeDtypeStruct((K, AUG, B), jnp.float32),
    )(cbt)
    out = pl.pallas_call(
        _pq_kernel,
        grid=(v // BV,),
        in_specs=[
            pl.BlockSpec((BV, K * SUBD), lambda i: (i, 0)),
            pl.BlockSpec((K, AUG, B), lambda i: (0, 0, 0)),
        ],
        out_specs=pl.BlockSpec((BV, K), lambda i: (i, 0)),
        out_shape=jax.ShapeDtypeStruct((v, K), jnp.uint8),
        compiler_params=pltpu.CompilerParams(
            dimension_semantics=("parallel",),
        ),
    )(vectors, cbs)
    return out
